# baseline (speedup 1.0000x reference)
import functools

import jax
import jax.numpy as jnp
from jax import lax
from jax.experimental import pallas as pl
from jax.experimental.pallas import tpu as pltpu
from jax.experimental.pallas import tpu_sc as plsc

NC = 2
NS = 16
L = 16
NW = NC * NS

DESC = 512
WORDS = DESC // 4
NL = 128
NPASS = 8
FP = 128
DPP = FP // 2


def _brief_body(tiles_per_w, x_hbm, ib_hbm, out_hbm,
                ibase, iw0, iw1, buf0, buf1, buf2, buf3, outw,
                sin0, sin1, sin2, sin3, sout):
    bufs = (buf0, buf1, buf2, buf3)
    iws = (iw0, iw1)
    sins = (sin0, sin1, sin2, sin3)
    wid = lax.axis_index("s") * NC + lax.axis_index("c")
    nt0 = wid * tiles_per_w

    pltpu.sync_copy(ib_hbm, ibase)

    iota = lax.iota(jnp.int32, L)
    obase = [(g * L + iota) * WORDS for g in range(NL // L)]

    def make_idx(t, iw):
        off = (nt0 + t) * 8

        def row(i, _):
            iw[i >> 3, pl.ds((i & 7) * L, L)] = ibase[pl.ds(i * L, L)] + off
            return 0

        lax.fori_loop(0, (NPASS * FP) // L, row, 0)

    def fetch(t, p):
        return pltpu.async_copy(
            x_hbm.at[iws[t % 2].at[p]], bufs[p % 4], sins[p % 4])

    def pass_compute(p, buf, out):
        def word_body(w, _):
            acc = [None] * (NL // L)
            for b in range(4):
                d = w * 4 + b
                bit = jnp.int32(1 << (8 * b))
                for g in range(NL // L):
                    v1 = buf[2 * d, pl.ds(g * L, L)]
                    v2 = buf[2 * d + 1, pl.ds(g * L, L)]
                    if b == 0:
                        acc[g] = jnp.where(v1 < v2, bit, jnp.int32(0))
                    else:
                        acc[g] = jnp.where(v1 < v2, acc[g] | bit, acc[g])
            m = p * (DPP // 4) + w
            for g in range(NL // L):
                plsc.store_scatter(out, [obase[g] + m], acc[g])
            return 0

        lax.fori_loop(0, DPP // 4, word_body, 0)

    make_idx(0, iws[0])
    in_dma = [None] * 4
    out_dma = None
    for t in range(tiles_per_w):
        for q in range(3):
            in_dma[q] = fetch(t, q)
        if t + 1 < tiles_per_w:
            make_idx(t + 1, iws[(t + 1) % 2])
        for p in range(NPASS):
            pb = p % 4
            if p + 3 < NPASS:
                in_dma[(p + 3) % 4] = fetch(t, p + 3)
            in_dma[pb].wait()
            if p == 0 and out_dma is not None:
                out_dma.wait()
            pass_compute(p, bufs[pb], outw)
        out_dma = pltpu.async_copy(
            outw, out_hbm.at[pl.ds((nt0 + t) * NL * WORDS, NL * WORDS)], sout)
    out_dma.wait()


def _unpack_body(w_ref, o_ref):
    w = w_ref[...]
    o_ref[...] = jnp.concatenate(
        [((w >> (8 * b)) & 1) == 1 for b in range(4)], axis=1)


def kernel(patches, pos1, pos2):
    n = patches.shape[0]
    assert n % (NW * NL) == 0
    tiles_per_w = n // (NW * NL)

    nt_cnt = n // NL
    r1 = pos1[:, 0].astype(jnp.int32)
    c1 = pos1[:, 1].astype(jnp.int32)
    r2 = pos2[:, 0].astype(jnp.int32)
    c2 = pos2[:, 1].astype(jnp.int32)
    b1 = (r1 * 4 + (c1 >> 3)) * (nt_cnt * 8) + (c1 & 7)
    b2 = (r2 * 4 + (c2 >> 3)) * (nt_cnt * 8) + (c2 & 7)
    b1 = b1.reshape(4, NL).T.reshape(DESC)
    b2 = b2.reshape(4, NL).T.reshape(DESC)
    ib = jnp.stack([b1, b2], axis=1).reshape(2 * DESC)

    xg = (patches.reshape(nt_cnt, NL, 32, 4, 8)
          .transpose(2, 3, 0, 4, 1)
          .reshape(nt_cnt * 1024, NL))

    mesh = plsc.VectorSubcoreMesh(core_axis_name="c", subcore_axis_name="s")
    body = functools.partial(_brief_body, tiles_per_w)
    out_words = pl.kernel(
        body,
        out_type=jax.ShapeDtypeStruct((n * WORDS,), jnp.int32),
        mesh=mesh,
        scratch_types=[
            pltpu.VMEM((2 * DESC,), jnp.int32),
            pltpu.VMEM((NPASS, FP), jnp.int32),
            pltpu.VMEM((NPASS, FP), jnp.int32),
            pltpu.VMEM((FP, NL), jnp.float32),
            pltpu.VMEM((FP, NL), jnp.float32),
            pltpu.VMEM((FP, NL), jnp.float32),
            pltpu.VMEM((FP, NL), jnp.float32),
            pltpu.VMEM((NL * WORDS,), jnp.int32),
            pltpu.SemaphoreType.DMA,
            pltpu.SemaphoreType.DMA,
            pltpu.SemaphoreType.DMA,
            pltpu.SemaphoreType.DMA,
            pltpu.SemaphoreType.DMA,
        ],
        compiler_params=pltpu.CompilerParams(needs_layout_passes=False),
    )(xg, ib)

    blk = 2048
    return pl.pallas_call(
        _unpack_body,
        out_shape=jax.ShapeDtypeStruct((n, DESC), jnp.bool_),
        grid=(n // blk,),
        in_specs=[pl.BlockSpec((blk, WORDS), lambda i: (i, 0))],
        out_specs=pl.BlockSpec((blk, DESC), lambda i: (i, 0)),
    )(out_words.reshape(n, WORDS))


# speedup vs baseline: 1.0408x; 1.0408x over previous
import functools

import jax
import jax.numpy as jnp
from jax import lax
from jax.experimental import pallas as pl
from jax.experimental.pallas import tpu as pltpu
from jax.experimental.pallas import tpu_sc as plsc

NC = 2
NS = 16
L = 16
NW = NC * NS

DESC = 512
WORDS = DESC // 4
NL = 128
NPASS = 8
FP = 128
DPP = FP // 2


def _brief_body(tiles_per_w, x_hbm, ib_hbm, out_hbm,
                ibase, iw0, iw1, buf0, buf1, outw,
                sin0, sin1, sout):
    bufs = (buf0, buf1)
    iws = (iw0, iw1)
    sins = (sin0, sin1)
    wid = lax.axis_index("s") * NC + lax.axis_index("c")
    nt0 = wid * tiles_per_w

    pltpu.sync_copy(ib_hbm, ibase)

    iota = lax.iota(jnp.int32, L)
    obase = [(g * L + iota) * WORDS for g in range(NL // L)]

    def make_idx(t, iw):
        off = (nt0 + t) * 8

        def row(i, _):
            iw[i >> 3, pl.ds((i & 7) * L, L)] = ibase[pl.ds(i * L, L)] + off
            return 0

        lax.fori_loop(0, (NPASS * FP) // L, row, 0)

    def fetch(t, p):
        return pltpu.async_copy(
            x_hbm.at[iws[t % 2].at[p]], bufs[p % 2], sins[p % 2])

    def pass_compute(p, buf, out):
        def word_body(w, _):
            acc = [None] * (NL // L)
            for b in range(4):
                d = w * 4 + b
                bit = jnp.int32(1 << (8 * b))
                for g in range(NL // L):
                    v1 = buf[2 * d, pl.ds(g * L, L)]
                    v2 = buf[2 * d + 1, pl.ds(g * L, L)]
                    if b == 0:
                        acc[g] = jnp.where(v1 < v2, bit, jnp.int32(0))
                    else:
                        acc[g] = jnp.where(v1 < v2, acc[g] | bit, acc[g])
            m = p * (DPP // 4) + w
            for g in range(NL // L):
                plsc.store_scatter(out, [obase[g] + m], acc[g])
            return 0

        lax.fori_loop(0, DPP // 4, word_body, 0)

    make_idx(0, iws[0])
    in_dma = [None, None]
    out_dma = None
    for t in range(tiles_per_w):
        in_dma[0] = fetch(t, 0)
        if t + 1 < tiles_per_w:
            make_idx(t + 1, iws[(t + 1) % 2])
        for p in range(NPASS):
            pb = p % 2
            if p + 1 < NPASS:
                in_dma[(pb + 1) % 2] = fetch(t, p + 1)
            in_dma[pb].wait()
            if p == 0 and out_dma is not None:
                out_dma.wait()
            pass_compute(p, bufs[pb], outw)
        out_dma = pltpu.async_copy(
            outw, out_hbm.at[pl.ds((nt0 + t) * NL * WORDS, NL * WORDS)], sout)
    out_dma.wait()


def _unpack_body(w_ref, o_ref):
    w = w_ref[...]
    o_ref[...] = jnp.concatenate(
        [((w >> (8 * b)) & 1) == 1 for b in range(4)], axis=1)


def kernel(patches, pos1, pos2):
    n = patches.shape[0]
    assert n % (NW * NL) == 0
    tiles_per_w = n // (NW * NL)

    nt_cnt = n // NL
    r1 = pos1[:, 0].astype(jnp.int32)
    c1 = pos1[:, 1].astype(jnp.int32)
    r2 = pos2[:, 0].astype(jnp.int32)
    c2 = pos2[:, 1].astype(jnp.int32)
    b1 = (r1 * 4 + (c1 >> 3)) * (nt_cnt * 8) + (c1 & 7)
    b2 = (r2 * 4 + (c2 >> 3)) * (nt_cnt * 8) + (c2 & 7)
    b1 = b1.reshape(4, NL).T.reshape(DESC)
    b2 = b2.reshape(4, NL).T.reshape(DESC)
    ib = jnp.stack([b1, b2], axis=1).reshape(2 * DESC)

    xg = (patches.reshape(nt_cnt, NL, 32, 4, 8)
          .transpose(2, 3, 0, 4, 1)
          .reshape(nt_cnt * 1024, NL))

    mesh = plsc.VectorSubcoreMesh(core_axis_name="c", subcore_axis_name="s")
    body = functools.partial(_brief_body, tiles_per_w)
    out_words = pl.kernel(
        body,
        out_type=jax.ShapeDtypeStruct((n * WORDS,), jnp.int32),
        mesh=mesh,
        scratch_types=[
            pltpu.VMEM((2 * DESC,), jnp.int32),
            pltpu.VMEM((NPASS, FP), jnp.int32),
            pltpu.VMEM((NPASS, FP), jnp.int32),
            pltpu.VMEM((FP, NL), jnp.float32),
            pltpu.VMEM((FP, NL), jnp.float32),
            pltpu.VMEM((NL * WORDS,), jnp.int32),
            pltpu.SemaphoreType.DMA,
            pltpu.SemaphoreType.DMA,
            pltpu.SemaphoreType.DMA,
        ],
        compiler_params=pltpu.CompilerParams(needs_layout_passes=False),
    )(xg, ib)

    blk = 2048
    return pl.pallas_call(
        _unpack_body,
        out_shape=jax.ShapeDtypeStruct((n, DESC), jnp.bool_),
        grid=(n // blk,),
        in_specs=[pl.BlockSpec((blk, WORDS), lambda i: (i, 0))],
        out_specs=pl.BlockSpec((blk, DESC), lambda i: (i, 0)),
    )(out_words.reshape(n, WORDS))
